# baseline (speedup 1.0000x reference)
import functools

import jax
import jax.numpy as jnp
from jax import lax
from jax.experimental import pallas as pl
from jax.experimental.pallas import tpu as pltpu
from jax.experimental.pallas import tpu_sc as plsc

N = 10000
E = 320000
H = 128

NC = 2
NS = 16
L = 16

NG = 2
BKT = N // NS
EG = E // NG
CK = 2000
NCH = EG // CK
GB = 64
CPAD = CK + GB
HK = H // L
NEG = -jnp.inf


def _enc_body(x_ref, ph_ref, we_ref, be_ref, wm_ref, bm_ref, z_ref, a2_ref, b_ref):
    xw = x_ref[...] * we_ref[0:1, :]
    z = jnp.dot(ph_ref[...], we_ref[1:, :], preferred_element_type=jnp.float32)
    z = jnp.maximum(z + xw + be_ref[...], 0.0)
    z_ref[...] = z
    a2_ref[...] = jnp.dot(z, wm_ref[:H, :], preferred_element_type=jnp.float32) + bm_ref[...]
    b_ref[...] = jnp.dot(z, wm_ref[H:2 * H, :], preferred_element_type=jnp.float32)


def _encode(x, pre_h, W_enc, b_enc, W_M, b_M):
    return pl.pallas_call(
        _enc_body,
        out_shape=[
            jax.ShapeDtypeStruct((N, H), jnp.float32),
            jax.ShapeDtypeStruct((N, H), jnp.float32),
            jax.ShapeDtypeStruct((N, H), jnp.float32),
        ],
    )(x, pre_h, W_enc, b_enc.reshape(1, H), W_M, b_M.reshape(1, H))


def _sc_body(b_hbm, e3_hbm, w_hbm, out_hbm,
             tab, eb, cli, csrc, catt, rows2, wv, sl0, sl1, sg0, sg1):
    s = lax.axis_index("s")
    g = lax.axis_index("c")
    nbase = s * BKT

    pltpu.sync_copy(w_hbm, wv)
    wregs = [wv[k, :] for k in range(HK)]

    def init(i, _):
        for k in range(HK):
            tab[i, pl.ds(k * L, L)] = jnp.full((L,), NEG, jnp.float32)
        return _
    lax.fori_loop(0, BKT, init, None)

    def zsrc(i, _):
        csrc[pl.ds(i * L, L)] = jnp.zeros((L,), jnp.int32)
        return _
    lax.fori_loop(0, CPAD // L, zsrc, None)

    lin_sems = (sl0, sl1)
    gat_sems = (sg0, sg1)

    def start_lin(ci, b):
        pltpu.async_copy(e3_hbm.at[:, pl.ds(g * EG + ci * CK, CK)],
                         eb.at[b], lin_sems[b])

    def start_gat(idx_off, q):
        pltpu.async_copy(b_hbm.at[csrc.at[pl.ds(idx_off, GB)]],
                         rows2.at[q], gat_sems[q])

    def do_edge(pos, e, rows):
        liv = cli[pl.ds(pos, L)]
        atv = catt[pl.ds(pos, L)]
        li = liv[0]
        a = plsc.bitcast(atv, jnp.float32)[0]
        for k in range(HK):
            val = rows[e, pl.ds(k * L, L)] + a * wregs[k]
            tab[li, pl.ds(k * L, L)] = jnp.maximum(tab[li, pl.ds(k * L, L)], val)

    def process(ci, b, start_next):
        pltpu.make_async_copy(e3_hbm.at[:, pl.ds(0, CK)], eb.at[b],
                              lin_sems[b]).wait()

        def scang(j, wp):
            s16 = eb[b, 0, pl.ds(j * L, L)]
            d16 = eb[b, 1, pl.ds(j * L, L)]
            a16 = eb[b, 2, pl.ds(j * L, L)]
            li = d16 - nbase
            ok = jnp.logical_and(li >= 0, li < BKT)
            plsc.store_compressed(cli.at[pl.ds(wp, L)], li, mask=ok)
            plsc.store_compressed(csrc.at[pl.ds(wp, L)], s16, mask=ok)
            plsc.store_compressed(catt.at[pl.ds(wp, L)], a16, mask=ok)
            return wp + plsc.all_reduce_population_count(ok)[0]
        cnt = lax.fori_loop(0, CK // L, scang, jnp.int32(0))

        if start_next:
            start_lin(ci + 1, 1 - b)

        nsb = (cnt + GB - 1) // GB

        @pl.when(nsb > 0)
        def _():
            start_gat(0, 0)

        def proc_sb(sbi, q):
            rem = jnp.minimum(cnt - sbi * GB, GB)
            pltpu.make_async_copy(b_hbm.at[pl.ds(0, GB)],
                                  rows2.at[q], gat_sems[q]).wait()

            def pe(e, _):
                do_edge(sbi * GB + e, e, rows2.at[q])
                return _
            lax.fori_loop(0, rem, pe, None)

        def sb2(k2, _):
            @pl.when(2 * k2 + 1 < nsb)
            def _():
                start_gat((2 * k2 + 1) * GB, 1)
            proc_sb(2 * k2, 0)

            @pl.when(2 * k2 + 2 < nsb)
            def _():
                start_gat((2 * k2 + 2) * GB, 0)

            @pl.when(2 * k2 + 1 < nsb)
            def _():
                proc_sb(2 * k2 + 1, 1)
            return _
        lax.fori_loop(0, (nsb + 1) // 2, sb2, None)

    start_lin(0, 0)

    def chunk_pair_main(p, _):
        process(2 * p, 0, True)
        process(2 * p + 1, 1, True)
        return _
    lax.fori_loop(0, NCH // 2 - 1, chunk_pair_main, None)
    process(NCH - 2, 0, True)
    process(NCH - 1, 1, False)

    pltpu.sync_copy(tab, out_hbm.at[g, s])


def _segmax(B, e3, w):
    mesh = plsc.VectorSubcoreMesh(core_axis_name="c", subcore_axis_name="s",
                                  num_cores=NC, num_subcores=NS)
    f = pl.kernel(
        _sc_body,
        out_type=jax.ShapeDtypeStruct((NG, NS, BKT, H), jnp.float32),
        mesh=mesh,
        compiler_params=pltpu.CompilerParams(use_tc_tiling_on_sc=False,
                                             needs_layout_passes=False),
        scratch_types=[
            pltpu.VMEM((BKT, H), jnp.float32),
            pltpu.VMEM((2, 3, CK), jnp.int32),
            pltpu.VMEM((CPAD,), jnp.int32),
            pltpu.VMEM((CPAD,), jnp.int32),
            pltpu.VMEM((CPAD,), jnp.int32),
            pltpu.VMEM((2, GB, H), jnp.float32),
            pltpu.VMEM((HK, L), jnp.float32),
            pltpu.SemaphoreType.DMA,
            pltpu.SemaphoreType.DMA,
            pltpu.SemaphoreType.DMA,
            pltpu.SemaphoreType.DMA,
        ],
    )
    return f(B, e3, w)


def _dec_body(p_ref, a2_ref, z_ref, wu_ref, bu_ref, wd_ref, bd_ref,
              wt_ref, bt_ref, h_ref, y_ref, t_ref):
    S = jnp.maximum(p_ref[0], p_ref[1])
    aggr = jnp.maximum(a2_ref[...] + S, 0.0)
    z = z_ref[...]
    h = jnp.dot(z, wu_ref[:H, :], preferred_element_type=jnp.float32)
    h = h + jnp.dot(aggr, wu_ref[H:, :], preferred_element_type=jnp.float32)
    h = jnp.maximum(h + bu_ref[...], 0.0)
    h_ref[...] = h
    y = jnp.dot(z, wd_ref[:H, :], preferred_element_type=jnp.float32)
    y = y + jnp.dot(h, wd_ref[H:, :], preferred_element_type=jnp.float32)
    y_ref[...] = jax.nn.sigmoid(y + bd_ref[...])
    hm = jnp.mean(h, axis=0, keepdims=True)
    wt = wt_ref[:H, :] + wt_ref[H:, :]
    t_ref[...] = jnp.dot(hm, wt, preferred_element_type=jnp.float32) + bt_ref[...]


def _decode(P, A2, z, W_U, b_U, W_dec, b_dec, W_ter, b_ter):
    return pl.pallas_call(
        _dec_body,
        out_shape=[
            jax.ShapeDtypeStruct((N, H), jnp.float32),
            jax.ShapeDtypeStruct((N, 1), jnp.float32),
            jax.ShapeDtypeStruct((1, 1), jnp.float32),
        ],
    )(P, A2, z, W_U, b_U.reshape(1, H), W_dec, b_dec.reshape(1, 1),
      W_ter, b_ter.reshape(1, 1))


def kernel(x, pre_h, edge_index, edge_attr, W_enc, b_enc, W_M, b_M,
           W_U, b_U, W_dec, b_dec, W_ter, b_ter):
    z, A2, B = _encode(x, pre_h, W_enc, b_enc, W_M, b_M)
    attr_i = lax.bitcast_convert_type(edge_attr[:, 0], jnp.int32)
    e3 = jnp.concatenate([edge_index, attr_i.reshape(1, E)], axis=0)
    w = W_M[2 * H].reshape(HK, L)
    P = _segmax(B, e3, w)
    P = P.reshape(NG, N, H)
    h, y, ter = _decode(P, A2, z, W_U, b_U, W_dec, b_dec, W_ter, b_ter)
    return (h, y, ter.reshape(()))


# speedup vs baseline: 2.1305x; 2.1305x over previous
import functools

import jax
import jax.numpy as jnp
from jax import lax
from jax.experimental import pallas as pl
from jax.experimental.pallas import tpu as pltpu
from jax.experimental.pallas import tpu_sc as plsc

N = 10000
E = 320000
H = 128

NC = 2
NS = 16
L = 16

NG = 2
BKT = N // NS
EG = E // NG
CK = 2000
NCH = EG // CK
GB = 32
CPAD = CK + GB
HK = H // L
NEG = -jnp.inf


def _enc_body(x_ref, ph_ref, we_ref, be_ref, wm_ref, bm_ref, z_ref, a2_ref, b_ref):
    xw = x_ref[...] * we_ref[0:1, :]
    z = jnp.dot(ph_ref[...], we_ref[1:, :], preferred_element_type=jnp.float32)
    z = jnp.maximum(z + xw + be_ref[...], 0.0)
    z_ref[...] = z
    a2_ref[...] = jnp.dot(z, wm_ref[:H, :], preferred_element_type=jnp.float32) + bm_ref[...]
    b_ref[...] = jnp.dot(z, wm_ref[H:2 * H, :], preferred_element_type=jnp.float32)


def _encode(x, pre_h, W_enc, b_enc, W_M, b_M):
    return pl.pallas_call(
        _enc_body,
        out_shape=[
            jax.ShapeDtypeStruct((N, H), jnp.float32),
            jax.ShapeDtypeStruct((N, H), jnp.float32),
            jax.ShapeDtypeStruct((N, H), jnp.float32),
        ],
    )(x, pre_h, W_enc, b_enc.reshape(1, H), W_M, b_M.reshape(1, H))


def _sc_body(b_hbm, e3_hbm, w_hbm, out_hbm,
             tab, eb, cli, csrc, catt, rows2, wv, sl0, sl1, sg0, sg1):
    s = lax.axis_index("s")
    g = lax.axis_index("c")
    nbase = s * BKT

    pltpu.sync_copy(w_hbm, wv)
    wregs = [wv[k, :] for k in range(HK)]

    def init(i, _):
        for k in range(HK):
            tab[i, pl.ds(k * L, L)] = jnp.full((L,), NEG, jnp.float32)
        return _
    lax.fori_loop(0, BKT, init, None)

    def zsrc(i, _):
        csrc[pl.ds(i * L, L)] = jnp.zeros((L,), jnp.int32)
        return _
    lax.fori_loop(0, CPAD // L, zsrc, None)

    lin_sems = (sl0, sl1)
    gat_sems = (sg0, sg1)

    def start_lin(ci, b):
        pltpu.async_copy(e3_hbm.at[:, pl.ds(g * EG + ci * CK, CK)],
                         eb.at[b], lin_sems[b])

    def start_gat(idx_off, q):
        pltpu.async_copy(b_hbm.at[csrc.at[pl.ds(idx_off, GB)]],
                         rows2.at[q], gat_sems[q])

    def do_edge(pos, e, rows):
        liv = cli[pl.ds(pos, L)]
        atv = catt[pl.ds(pos, L)]
        li = liv[0]
        a = plsc.bitcast(atv, jnp.float32)[0]
        for k in range(HK):
            val = rows[e, pl.ds(k * L, L)] + a * wregs[k]
            tab[li, pl.ds(k * L, L)] = jnp.maximum(tab[li, pl.ds(k * L, L)], val)

    def process(ci, b, start_next):
        pltpu.make_async_copy(e3_hbm.at[:, pl.ds(0, CK)], eb.at[b],
                              lin_sems[b]).wait()

        def scang(j, wp):
            s16 = eb[b, 0, pl.ds(j * L, L)]
            d16 = eb[b, 1, pl.ds(j * L, L)]
            a16 = eb[b, 2, pl.ds(j * L, L)]
            li = d16 - nbase
            ok = jnp.logical_and(li >= 0, li < BKT)
            plsc.store_compressed(cli.at[pl.ds(wp, L)], li, mask=ok)
            plsc.store_compressed(csrc.at[pl.ds(wp, L)], s16, mask=ok)
            plsc.store_compressed(catt.at[pl.ds(wp, L)], a16, mask=ok)
            return wp + plsc.all_reduce_population_count(ok)[0]
        cnt = lax.fori_loop(0, CK // L, scang, jnp.int32(0))

        if start_next:
            start_lin(ci + 1, 1 - b)

        nsb = (cnt + GB - 1) // GB

        @pl.when(nsb > 0)
        def _():
            start_gat(0, 0)

        def proc_sb(sbi, q):
            rem = jnp.minimum(cnt - sbi * GB, GB)
            pltpu.make_async_copy(b_hbm.at[pl.ds(0, GB)],
                                  rows2.at[q], gat_sems[q]).wait()

            def pe2(i2, _):
                do_edge(sbi * GB + 2 * i2, 2 * i2, rows2.at[q])
                do_edge(sbi * GB + 2 * i2 + 1, 2 * i2 + 1, rows2.at[q])
                return _
            lax.fori_loop(0, rem // 2, pe2, None)

            @pl.when(rem % 2 == 1)
            def _():
                do_edge(sbi * GB + rem - 1, rem - 1, rows2.at[q])

        def sb2(k2, _):
            @pl.when(2 * k2 + 1 < nsb)
            def _():
                start_gat((2 * k2 + 1) * GB, 1)
            proc_sb(2 * k2, 0)

            @pl.when(2 * k2 + 2 < nsb)
            def _():
                start_gat((2 * k2 + 2) * GB, 0)

            @pl.when(2 * k2 + 1 < nsb)
            def _():
                proc_sb(2 * k2 + 1, 1)
            return _
        lax.fori_loop(0, (nsb + 1) // 2, sb2, None)

    start_lin(0, 0)

    def chunk_pair_main(p, _):
        process(2 * p, 0, True)
        process(2 * p + 1, 1, True)
        return _
    lax.fori_loop(0, NCH // 2 - 1, chunk_pair_main, None)
    process(NCH - 2, 0, True)
    process(NCH - 1, 1, False)

    pltpu.sync_copy(tab, out_hbm.at[g, s])


def _segmax(B, e3, w):
    mesh = plsc.VectorSubcoreMesh(core_axis_name="c", subcore_axis_name="s",
                                  num_cores=NC, num_subcores=NS)
    f = pl.kernel(
        _sc_body,
        out_type=jax.ShapeDtypeStruct((NG, NS, BKT, H), jnp.float32),
        mesh=mesh,
        compiler_params=pltpu.CompilerParams(use_tc_tiling_on_sc=False,
                                             needs_layout_passes=False),
        scratch_types=[
            pltpu.VMEM((BKT, H), jnp.float32),
            pltpu.VMEM((2, 3, CK), jnp.int32),
            pltpu.VMEM((CPAD,), jnp.int32),
            pltpu.VMEM((CPAD,), jnp.int32),
            pltpu.VMEM((CPAD,), jnp.int32),
            pltpu.VMEM((2, GB, H), jnp.float32),
            pltpu.VMEM((HK, L), jnp.float32),
            pltpu.SemaphoreType.DMA,
            pltpu.SemaphoreType.DMA,
            pltpu.SemaphoreType.DMA,
            pltpu.SemaphoreType.DMA,
        ],
    )
    return f(B, e3, w)


def _dec_body(p_ref, a2_ref, z_ref, wu_ref, bu_ref, wd_ref, bd_ref,
              wt_ref, bt_ref, h_ref, y_ref, t_ref):
    S = jnp.maximum(p_ref[0], p_ref[1])
    aggr = jnp.maximum(a2_ref[...] + S, 0.0)
    z = z_ref[...]
    h = jnp.dot(z, wu_ref[:H, :], preferred_element_type=jnp.float32)
    h = h + jnp.dot(aggr, wu_ref[H:, :], preferred_element_type=jnp.float32)
    h = jnp.maximum(h + bu_ref[...], 0.0)
    h_ref[...] = h
    y = jnp.dot(z, wd_ref[:H, :], preferred_element_type=jnp.float32)
    y = y + jnp.dot(h, wd_ref[H:, :], preferred_element_type=jnp.float32)
    y_ref[...] = jax.nn.sigmoid(y + bd_ref[...])
    hm = jnp.mean(h, axis=0, keepdims=True)
    wt = wt_ref[:H, :] + wt_ref[H:, :]
    t_ref[...] = jnp.dot(hm, wt, preferred_element_type=jnp.float32) + bt_ref[...]


def _decode(P, A2, z, W_U, b_U, W_dec, b_dec, W_ter, b_ter):
    return pl.pallas_call(
        _dec_body,
        out_shape=[
            jax.ShapeDtypeStruct((N, H), jnp.float32),
            jax.ShapeDtypeStruct((N, 1), jnp.float32),
            jax.ShapeDtypeStruct((1, 1), jnp.float32),
        ],
    )(P, A2, z, W_U, b_U.reshape(1, H), W_dec, b_dec.reshape(1, 1),
      W_ter, b_ter.reshape(1, 1))


def kernel(x, pre_h, edge_index, edge_attr, W_enc, b_enc, W_M, b_M,
           W_U, b_U, W_dec, b_dec, W_ter, b_ter):
    z, A2, B = _encode(x, pre_h, W_enc, b_enc, W_M, b_M)
    attr_i = lax.bitcast_convert_type(edge_attr[:, 0], jnp.int32)
    e3 = jnp.concatenate([edge_index, attr_i.reshape(1, E)], axis=0)
    w = W_M[2 * H].reshape(HK, L)
    P = _segmax(B, e3, w)
    P = P.reshape(NG, N, H)
    h, y, ter = _decode(P, A2, z, W_U, b_U, W_dec, b_dec, W_ter, b_ter)
    return (h, y, ter.reshape(()))


# speedup vs baseline: 2.3379x; 1.0974x over previous
import functools

import jax
import jax.numpy as jnp
from jax import lax
from jax.experimental import pallas as pl
from jax.experimental.pallas import tpu as pltpu
from jax.experimental.pallas import tpu_sc as plsc

N = 10000
E = 320000
H = 128

NC = 2
NS = 16
L = 16

NG = 2
BKT = N // NS
EG = E // NG
CK = 2000
NCH = EG // CK
GB = 16
CPAD = CK + GB
HK = H // L
NEG = -jnp.inf


def _enc_body(x_ref, ph_ref, we_ref, be_ref, wm_ref, bm_ref, z_ref, a2_ref, b_ref):
    xw = x_ref[...] * we_ref[0:1, :]
    z = jnp.dot(ph_ref[...], we_ref[1:, :], preferred_element_type=jnp.float32)
    z = jnp.maximum(z + xw + be_ref[...], 0.0)
    z_ref[...] = z
    a2_ref[...] = jnp.dot(z, wm_ref[:H, :], preferred_element_type=jnp.float32) + bm_ref[...]
    b_ref[...] = jnp.dot(z, wm_ref[H:2 * H, :], preferred_element_type=jnp.float32)


def _encode(x, pre_h, W_enc, b_enc, W_M, b_M):
    return pl.pallas_call(
        _enc_body,
        out_shape=[
            jax.ShapeDtypeStruct((N, H), jnp.float32),
            jax.ShapeDtypeStruct((N, H), jnp.float32),
            jax.ShapeDtypeStruct((N, H), jnp.float32),
        ],
    )(x, pre_h, W_enc, b_enc.reshape(1, H), W_M, b_M.reshape(1, H))


def _sc_body(b_hbm, e3_hbm, w_hbm, out_hbm,
             tab, eb, cli, csrc, catt, rows2, wv, sl0, sl1, sg0, sg1):
    s = lax.axis_index("s")
    g = lax.axis_index("c")
    nbase = s * BKT

    pltpu.sync_copy(w_hbm, wv)
    wregs = [wv[k, :] for k in range(HK)]

    def init(i, _):
        for k in range(HK):
            tab[i, pl.ds(k * L, L)] = jnp.full((L,), NEG, jnp.float32)
        return _
    lax.fori_loop(0, BKT, init, None)

    def zsrc(i, _):
        csrc[pl.ds(i * L, L)] = jnp.zeros((L,), jnp.int32)
        return _
    lax.fori_loop(0, CPAD // L, zsrc, None)

    lin_sems = (sl0, sl1)
    gat_sems = (sg0, sg1)

    def start_lin(ci, b):
        pltpu.async_copy(e3_hbm.at[:, pl.ds(g * EG + ci * CK, CK)],
                         eb.at[b], lin_sems[b])

    def start_gat(idx_off, q):
        pltpu.async_copy(b_hbm.at[csrc.at[pl.ds(idx_off, GB)]],
                         rows2.at[q], gat_sems[q])

    def do_edge(pos, e, rows):
        liv = cli[pl.ds(pos, L)]
        atv = catt[pl.ds(pos, L)]
        li = liv[0]
        a = plsc.bitcast(atv, jnp.float32)[0]
        for k in range(HK):
            val = rows[e, pl.ds(k * L, L)] + a * wregs[k]
            tab[li, pl.ds(k * L, L)] = jnp.maximum(tab[li, pl.ds(k * L, L)], val)

    def process(ci, b, start_next):
        pltpu.make_async_copy(e3_hbm.at[:, pl.ds(0, CK)], eb.at[b],
                              lin_sems[b]).wait()

        def scang(j, wp):
            s16 = eb[b, 0, pl.ds(j * L, L)]
            d16 = eb[b, 1, pl.ds(j * L, L)]
            a16 = eb[b, 2, pl.ds(j * L, L)]
            li = d16 - nbase
            ok = jnp.logical_and(li >= 0, li < BKT)
            plsc.store_compressed(cli.at[pl.ds(wp, L)], li, mask=ok)
            plsc.store_compressed(csrc.at[pl.ds(wp, L)], s16, mask=ok)
            plsc.store_compressed(catt.at[pl.ds(wp, L)], a16, mask=ok)
            return wp + plsc.all_reduce_population_count(ok)[0]
        cnt = lax.fori_loop(0, CK // L, scang, jnp.int32(0))

        if start_next:
            start_lin(ci + 1, 1 - b)

        nsb = (cnt + GB - 1) // GB

        @pl.when(nsb > 0)
        def _():
            start_gat(0, 0)

        def proc_sb(sbi, q):
            rem = jnp.minimum(cnt - sbi * GB, GB)
            pltpu.make_async_copy(b_hbm.at[pl.ds(0, GB)],
                                  rows2.at[q], gat_sems[q]).wait()

            def pe(e, _):
                do_edge(sbi * GB + e, e, rows2.at[q])
                return _
            lax.fori_loop(0, rem, pe, None)

        def sb2(k2, _):
            @pl.when(2 * k2 + 1 < nsb)
            def _():
                start_gat((2 * k2 + 1) * GB, 1)
            proc_sb(2 * k2, 0)

            @pl.when(2 * k2 + 2 < nsb)
            def _():
                start_gat((2 * k2 + 2) * GB, 0)

            @pl.when(2 * k2 + 1 < nsb)
            def _():
                proc_sb(2 * k2 + 1, 1)
            return _
        lax.fori_loop(0, (nsb + 1) // 2, sb2, None)

    start_lin(0, 0)

    def chunk_pair_main(p, _):
        process(2 * p, 0, True)
        process(2 * p + 1, 1, True)
        return _
    lax.fori_loop(0, NCH // 2 - 1, chunk_pair_main, None)
    process(NCH - 2, 0, True)
    process(NCH - 1, 1, False)

    pltpu.sync_copy(tab, out_hbm.at[g, s])


def _segmax(B, e3, w):
    mesh = plsc.VectorSubcoreMesh(core_axis_name="c", subcore_axis_name="s",
                                  num_cores=NC, num_subcores=NS)
    f = pl.kernel(
        _sc_body,
        out_type=jax.ShapeDtypeStruct((NG, NS, BKT, H), jnp.float32),
        mesh=mesh,
        compiler_params=pltpu.CompilerParams(use_tc_tiling_on_sc=False,
                                             needs_layout_passes=False),
        scratch_types=[
            pltpu.VMEM((BKT, H), jnp.float32),
            pltpu.VMEM((2, 3, CK), jnp.int32),
            pltpu.VMEM((CPAD,), jnp.int32),
            pltpu.VMEM((CPAD,), jnp.int32),
            pltpu.VMEM((CPAD,), jnp.int32),
            pltpu.VMEM((2, GB, H), jnp.float32),
            pltpu.VMEM((HK, L), jnp.float32),
            pltpu.SemaphoreType.DMA,
            pltpu.SemaphoreType.DMA,
            pltpu.SemaphoreType.DMA,
            pltpu.SemaphoreType.DMA,
        ],
    )
    return f(B, e3, w)


def _dec_body(p_ref, a2_ref, z_ref, wu_ref, bu_ref, wd_ref, bd_ref,
              wt_ref, bt_ref, h_ref, y_ref, t_ref):
    S = jnp.maximum(p_ref[0], p_ref[1])
    aggr = jnp.maximum(a2_ref[...] + S, 0.0)
    z = z_ref[...]
    h = jnp.dot(z, wu_ref[:H, :], preferred_element_type=jnp.float32)
    h = h + jnp.dot(aggr, wu_ref[H:, :], preferred_element_type=jnp.float32)
    h = jnp.maximum(h + bu_ref[...], 0.0)
    h_ref[...] = h
    y = jnp.dot(z, wd_ref[:H, :], preferred_element_type=jnp.float32)
    y = y + jnp.dot(h, wd_ref[H:, :], preferred_element_type=jnp.float32)
    y_ref[...] = jax.nn.sigmoid(y + bd_ref[...])
    hm = jnp.mean(h, axis=0, keepdims=True)
    wt = wt_ref[:H, :] + wt_ref[H:, :]
    t_ref[...] = jnp.dot(hm, wt, preferred_element_type=jnp.float32) + bt_ref[...]


def _decode(P, A2, z, W_U, b_U, W_dec, b_dec, W_ter, b_ter):
    return pl.pallas_call(
        _dec_body,
        out_shape=[
            jax.ShapeDtypeStruct((N, H), jnp.float32),
            jax.ShapeDtypeStruct((N, 1), jnp.float32),
            jax.ShapeDtypeStruct((1, 1), jnp.float32),
        ],
    )(P, A2, z, W_U, b_U.reshape(1, H), W_dec, b_dec.reshape(1, 1),
      W_ter, b_ter.reshape(1, 1))


def kernel(x, pre_h, edge_index, edge_attr, W_enc, b_enc, W_M, b_M,
           W_U, b_U, W_dec, b_dec, W_ter, b_ter):
    z, A2, B = _encode(x, pre_h, W_enc, b_enc, W_M, b_M)
    attr_i = lax.bitcast_convert_type(edge_attr[:, 0], jnp.int32)
    e3 = jnp.concatenate([edge_index, attr_i.reshape(1, E)], axis=0)
    w = W_M[2 * H].reshape(HK, L)
    P = _segmax(B, e3, w)
    P = P.reshape(NG, N, H)
    h, y, ter = _decode(P, A2, z, W_U, b_U, W_dec, b_dec, W_ter, b_ter)
    return (h, y, ter.reshape(()))
